# baseline (speedup 1.0000x reference)
import functools
import math

import jax
import jax.numpy as jnp
from jax import lax
from jax.experimental import pallas as pl
from jax.experimental.pallas import tpu as pltpu
from jax.experimental.pallas import tpu_sc as plsc

D_MODEL = 64
SCALE = math.sqrt(D_MODEL)

NUM_CORES = 2
NUM_SUBCORES = 16
LANES = 16
NW = NUM_CORES * NUM_SUBCORES

NBUF = 2


@functools.lru_cache(maxsize=None)
def _build_sc_gather(S0: int, S1: int, V: int):
    assert S0 % (NW * 128) == 0 or S0 == NW * 128
    lanes_per_w = S0 // NW

    mesh = plsc.VectorSubcoreMesh(core_axis_name="c", subcore_axis_name="s")

    @functools.partial(
        pl.kernel,
        out_type=jax.ShapeDtypeStruct((S1, D_MODEL, S0), jnp.float32),
        mesh=mesh,
        scratch_types=[
            pltpu.VMEM((S1, lanes_per_w), jnp.int32),
            pltpu.VMEM((NBUF, lanes_per_w), jnp.int32),
            pltpu.VMEM((NBUF, lanes_per_w, 128), jnp.float32),
            pltpu.VMEM((NBUF, D_MODEL, lanes_per_w), jnp.float32),
            pltpu.SemaphoreType.DMA,
            pltpu.SemaphoreType.DMA,
        ],
        compiler_params=pltpu.CompilerParams(needs_layout_passes=False),
    )
    def k(xt_hbm, tab_hbm, out_hbm, idx_v, pb_v, rows_v, ob_v, gsem, wsem):
        wid = lax.axis_index("s") * NUM_CORES + lax.axis_index("c")
        base = wid * lanes_per_w
        pltpu.sync_copy(xt_hbm.at[:, pl.ds(base, lanes_per_w)], idx_v)

        def compute_p(g, slot):
            for kk in range(lanes_per_w // LANES):
                sl = pl.ds(kk * LANES, LANES)
                pb_v[slot, sl] = jnp.right_shift(idx_v[g, sl], 1)

        def start_gather(slot):
            pltpu.async_copy(tab_hbm.at[pb_v.at[slot]], rows_v.at[slot], gsem)

        def wait_gather(slot):
            pltpu.make_async_copy(
                tab_hbm.at[pb_v.at[slot]], rows_v.at[slot], gsem
            ).wait()

        def start_write(g, slot):
            pltpu.async_copy(
                ob_v.at[slot], out_hbm.at[g, :, pl.ds(base, lanes_per_w)], wsem
            )

        def wait_write(slot):
            pltpu.make_async_copy(
                ob_v.at[slot], out_hbm.at[0, :, pl.ds(base, lanes_per_w)], wsem
            ).wait()

        def compute_out(g, slot):
            rows = rows_v.at[slot]
            for kk in range(lanes_per_w // LANES):
                sl = pl.ds(kk * LANES, LANES)
                row_id = jax.lax.iota(jnp.int32, LANES) + kk * LANES
                half = jnp.left_shift(jnp.bitwise_and(idx_v[g, sl], 1), 6)
                for f in range(D_MODEL):
                    col = half + f
                    vals = plsc.load_gather(rows, [row_id, col])
                    ob_v[slot, f, sl] = vals * SCALE

        assert S1 % NBUF == 0
        compute_p(0, 0)
        start_gather(0)
        compute_p(1, 1)

        @pl.loop(0, S1, step=NBUF)
        def _(g0):
            for b in range(NBUF):
                g = g0 + b
                nxt = g + 1

                @pl.when(g >= NBUF)
                def _():
                    wait_write(b)

                @pl.when(nxt < S1)
                def _():
                    start_gather((b + 1) % NBUF)

                wait_gather(b)
                compute_out(g, b)

                @pl.when(nxt + 1 < S1)
                def _():
                    compute_p(nxt + 1, b)

                start_write(g, b)

        wait_write(0)
        wait_write(1)

    return k


def kernel(x, lut):
    S0, S1 = x.shape
    V = lut.shape[0]
    tab = lut.reshape(V // 2, 2 * D_MODEL)
    k = _build_sc_gather(S0, S1, V)
    out = k(x.T, tab)
    return out.transpose(2, 0, 1)


# speedup vs baseline: 1.7535x; 1.7535x over previous
import functools
import math

import jax
import jax.numpy as jnp
from jax import lax
from jax.experimental import pallas as pl
from jax.experimental.pallas import tpu as pltpu
from jax.experimental.pallas import tpu_sc as plsc

D_MODEL = 64
SCALE = math.sqrt(D_MODEL)

NUM_CORES = 2
NUM_SUBCORES = 16
LANES = 16
NW = NUM_CORES * NUM_SUBCORES

NBUF = 2


@functools.lru_cache(maxsize=None)
def _build_sc_gather(S0: int, S1: int, V: int):
    assert S0 % (NW * 128) == 0 or S0 == NW * 128
    lanes_per_w = S0 // NW

    mesh = plsc.VectorSubcoreMesh(core_axis_name="c", subcore_axis_name="s")

    @functools.partial(
        pl.kernel,
        out_type=jax.ShapeDtypeStruct((S1, D_MODEL, S0), jnp.float32),
        mesh=mesh,
        scratch_types=[
            pltpu.VMEM((S1, lanes_per_w), jnp.int32),
            pltpu.VMEM((NBUF, lanes_per_w), jnp.int32),
            pltpu.VMEM((NBUF, lanes_per_w, 128), jnp.float32),
            pltpu.VMEM((NBUF, D_MODEL, 130), jnp.float32),
            pltpu.SemaphoreType.DMA,
            pltpu.SemaphoreType.DMA,
        ],
        compiler_params=pltpu.CompilerParams(needs_layout_passes=False),
    )
    def k(xt_hbm, tab_hbm, out_hbm, idx_v, pb_v, rows_v, ob_v, gsem, wsem):
        wid = lax.axis_index("s") * NUM_CORES + lax.axis_index("c")
        base = wid * lanes_per_w
        pltpu.sync_copy(xt_hbm.at[:, pl.ds(base, lanes_per_w)], idx_v)

        def compute_p(g, slot):
            for kk in range(lanes_per_w // LANES):
                sl = pl.ds(kk * LANES, LANES)
                pb_v[slot, sl] = jnp.right_shift(idx_v[g, sl], 1)

        def start_gather(slot):
            pltpu.async_copy(tab_hbm.at[pb_v.at[slot]], rows_v.at[slot], gsem)

        def wait_gather(slot):
            pltpu.make_async_copy(
                tab_hbm.at[pb_v.at[slot]], rows_v.at[slot], gsem
            ).wait()

        def start_write(g, slot):
            pltpu.async_copy(
                ob_v.at[slot, :, pl.ds(0, lanes_per_w)],
                out_hbm.at[g, :, pl.ds(base, lanes_per_w)],
                wsem,
            )

        def wait_write(slot):
            pltpu.make_async_copy(
                ob_v.at[slot, :, pl.ds(0, lanes_per_w)],
                out_hbm.at[0, :, pl.ds(base, lanes_per_w)],
                wsem,
            ).wait()

        iota16 = jax.lax.iota(jnp.int32, LANES)

        def compute_out(g, slot):
            ob = ob_v.at[slot]
            rows = rows_v.at[slot]

            @pl.loop(0, lanes_per_w // LANES)
            def _(kk):
                tok = iota16 + kk * LANES
                hv = jnp.left_shift(
                    jnp.bitwise_and(idx_v[g, pl.ds(kk * LANES, LANES)], 1), 6
                )
                for d in range(LANES):
                    fbase = jnp.bitwise_and(iota16 + d, LANES - 1)
                    cbase = hv + fbase
                    for j in range(D_MODEL // LANES):
                        frow = fbase + j * LANES
                        vals = plsc.load_gather(rows, [tok, cbase + j * LANES])
                        plsc.store_scatter(ob, [frow, tok], vals * SCALE)

        assert S1 % NBUF == 0
        compute_p(0, 0)
        start_gather(0)
        compute_p(1, 1)

        @pl.loop(0, S1, step=NBUF)
        def _(g0):
            for b in range(NBUF):
                g = g0 + b
                nxt = g + 1

                @pl.when(g >= NBUF)
                def _():
                    wait_write(b)

                @pl.when(nxt < S1)
                def _():
                    start_gather((b + 1) % NBUF)

                wait_gather(b)
                compute_out(g, b)

                @pl.when(nxt + 1 < S1)
                def _():
                    compute_p(nxt + 1, b)

                start_write(g, b)

        wait_write(0)
        wait_write(1)

    return k


def kernel(x, lut):
    S0, S1 = x.shape
    V = lut.shape[0]
    tab = lut.reshape(V // 2, 2 * D_MODEL)
    k = _build_sc_gather(S0, S1, V)
    out = k(x.T, tab)
    return out.transpose(2, 0, 1)
